# baseline (speedup 1.0000x reference)
import functools

import jax
import jax.numpy as jnp
from jax import lax
from jax.experimental import pallas as pl
from jax.experimental.pallas import tpu as pltpu
from jax.experimental.pallas import tpu_sc as plsc

N = 50000
E = 80000
P = 25600
F = 20
H = 128

NP_ = 50176
EP = 81920
SUB = 128
TILE_E = EP // 16
NSUB = TILE_E // SUB
HALF = NP_ // 2
ACC_R = HALF + 64
TILE_R = HALF // 16
ZR = 784

_mesh = functools.partial(
    plsc.VectorSubcoreMesh, core_axis_name="c", subcore_axis_name="s")


def _seg_sum_sc(table, src_flat, dst_flat, n_real, n_chunks, n_cnt=0,
                w=16):
  @functools.partial(
      pl.kernel, mesh=_mesh(),
      compiler_params=pltpu.CompilerParams(use_tc_tiling_on_sc=False),
      out_type=jax.ShapeDtypeStruct(((n_real + n_cnt) * n_chunks * NP_, w),
                                    jnp.float32),
      scratch_types=[
          pltpu.VMEM((SUB,), jnp.int32),
          pltpu.VMEM((SUB,), jnp.int32),
          pltpu.VMEM((SUB,), jnp.int32),
          pltpu.VMEM((SUB,), jnp.int32),
          pltpu.VMEM((SUB, w), jnp.float32),
          pltpu.VMEM((SUB, w), jnp.float32),
          pltpu.VMEM((ZR, w), jnp.float32),
          pltpu.VMEM((ZR, w), jnp.float32),
          pltpu.SemaphoreType.DMA,
          pltpu.SemaphoreType.DMA,
          pltpu.VMEM_SHARED((ACC_R, w), jnp.float32),
      ])
  def seg(table_hbm, src_hbm, dst_hbm, zero_hbm, out_hbm,
          src_v0, src_v1, dst_v0, dst_v1, rows_v0, rows_v1,
          zero_v, drain_v, sem0, sem1, acc):
    c = lax.axis_index("c")
    s = lax.axis_index("s")
    pltpu.sync_copy(zero_hbm, zero_v)
    row0 = s * TILE_R
    half0 = c * HALF
    srcs = (src_v0, src_v1)
    dsts = (dst_v0, dst_v1)
    rows = (rows_v0, rows_v1)
    sems = (sem0, sem1)
    for chunk in range(n_chunks):
      tab_base = chunk * NP_
      n_t = n_real + n_cnt if chunk == 0 else n_real

      def tbody(t, _):
        for r in range(TILE_R // ZR):
          pltpu.sync_copy(zero_v, acc.at[pl.ds(row0 + r * ZR, ZR)])
        plsc.subcore_barrier()
        ebase0 = t * EP + s * TILE_E

        def ebody(jj, _):
          cps = []
          for b in range(2):
            eb = ebase0 + (jj * 2 + b) * SUB
            pltpu.sync_copy(src_hbm.at[pl.ds(eb, SUB)], srcs[b])
            pltpu.sync_copy(dst_hbm.at[pl.ds(eb, SUB)], dsts[b])
            for q in range(SUB // 16):
              sl = pl.ds(q * 16, 16)
              if chunk:
                srcs[b][sl] = srcs[b][sl] + tab_base
              dl = dsts[b][sl] - half0
              ok = (dl >= 0) & (dl < HALF)
              dsts[b][sl] = jnp.where(ok, dl, HALF)
            cps.append(pltpu.async_copy(table_hbm.at[srcs[b]], rows[b],
                                        sems[b]))
          for b in range(2):
            cps[b].wait()
            pltpu.sync_copy(rows[b], acc.at[dsts[b]], add=True)
          return 0

        lax.fori_loop(0, NSUB // 2, ebody, 0)
        plsc.subcore_barrier()
        obase = (t * n_chunks + chunk) * NP_ + half0 + row0
        for r in range(TILE_R // ZR):
          pltpu.sync_copy(acc.at[pl.ds(row0 + r * ZR, ZR)], drain_v)
          pltpu.sync_copy(drain_v, out_hbm.at[pl.ds(obase + r * ZR, ZR)])
        plsc.subcore_barrier()
        return 0

      lax.fori_loop(0, n_t, tbody, 0)

  zero_hbm = jnp.zeros((ZR, w), jnp.float32)
  return seg(table, src_flat, dst_flat, zero_hbm)


def _gather_sc(table, idx, n_rows, gsub):
  bpw = n_rows // 32
  D = table.shape[1]

  @functools.partial(
      pl.kernel, mesh=_mesh(),
      compiler_params=pltpu.CompilerParams(use_tc_tiling_on_sc=False),
      out_type=jax.ShapeDtypeStruct((n_rows, D), jnp.float32),
      scratch_types=[
          pltpu.VMEM((gsub,), jnp.int32),
          pltpu.VMEM((gsub, D), jnp.float32),
          pltpu.SemaphoreType.DMA,
      ])
  def gat(table_hbm, idx_hbm, out_hbm, idx_v, rows_v, sem):
    c = lax.axis_index("c")
    s = lax.axis_index("s")
    wid = s * 2 + c
    base = wid * bpw

    def body(j, _):
      b = base + j * gsub
      pltpu.sync_copy(idx_hbm.at[pl.ds(b, gsub)], idx_v)
      pltpu.async_copy(table_hbm.at[idx_v], rows_v, sem).wait()
      pltpu.sync_copy(rows_v, out_hbm.at[pl.ds(b, gsub)])
      return 0

    lax.fori_loop(0, bpw // gsub, body, 0)

  return gat(table, idx)


BLK = 256


def _embed_body(x_ref, hp_ref, hs_ref, we_ref, be_ref, o_ref):
  xb = x_ref[...]
  hc = jnp.dot(xb[:, 2:F - 1], we_ref[...],
               preferred_element_type=jnp.float32) + be_ref[...]
  o_ref[...] = jnp.concatenate([hc, hp_ref[...], hs_ref[...]], axis=-1)


def _embed_tc(x_pad, hp, hs, W_emb, b_emb):
  wet = W_emb.T.astype(jnp.float32)
  be = b_emb.reshape(1, 32).astype(jnp.float32)
  grid = (NP_ // BLK,)
  return pl.pallas_call(
      _embed_body,
      grid=grid,
      in_specs=[
          pl.BlockSpec((BLK, F), lambda i: (i, 0)),
          pl.BlockSpec((BLK, 16), lambda i: (i, 0)),
          pl.BlockSpec((BLK, 16), lambda i: (i, 0)),
          pl.BlockSpec((F - 3, 32), lambda i: (0, 0)),
          pl.BlockSpec((1, 32), lambda i: (0, 0)),
      ],
      out_specs=pl.BlockSpec((BLK, 64), lambda i: (i, 0)),
      out_shape=jax.ShapeDtypeStruct((NP_, 64), jnp.float32),
  )(x_pad, hp, hs, wet, be)


def _layer_body(s_ref, cnt_ref, h_ref, w_ref, b_ref, o_ref, *,
                n_types, d):
  sblk = s_ref[...]
  blk = sblk.shape[0]
  inv = 1.0 / jnp.maximum(cnt_ref[...], 1.0)
  m = sblk.reshape(blk, n_types, d) * inv[:, :, None]
  h = h_ref[...]
  hrep = jnp.broadcast_to(h[:, None, :], (blk, n_types, d))
  maug = jnp.concatenate([m, hrep], axis=1).reshape(blk, 2 * n_types * d)
  acc = jnp.dot(maug, w_ref[...], preferred_element_type=jnp.float32)
  o_ref[...] = jax.nn.relu((acc + b_ref[...]) / float(n_types))


def _layer_tc(s_cat, cnt_t, h_in, Wn, Wr, b):
  n_types = Wn.shape[0]
  d = h_in.shape[1]
  w_aug = jnp.concatenate([
      Wn.transpose(0, 2, 1).reshape(n_types * d, H),
      Wr.transpose(0, 2, 1).reshape(n_types * d, H)], axis=0)
  b_eff = b.sum(0).reshape(1, H).astype(jnp.float32)
  grid = (NP_ // BLK,)
  return pl.pallas_call(
      functools.partial(_layer_body, n_types=n_types, d=d),
      grid=grid,
      in_specs=[
          pl.BlockSpec((BLK, n_types * d), lambda i: (i, 0)),
          pl.BlockSpec((BLK, n_types), lambda i: (i, 0)),
          pl.BlockSpec((BLK, d), lambda i: (i, 0)),
          pl.BlockSpec((2 * n_types * d, H), lambda i: (0, 0)),
          pl.BlockSpec((1, H), lambda i: (0, 0)),
      ],
      out_specs=pl.BlockSpec((BLK, H), lambda i: (i, 0)),
      out_shape=jax.ShapeDtypeStruct((NP_, H), jnp.float32),
  )(s_cat, cnt_t, h_in, w_aug.astype(jnp.float32), b_eff)


def _norm_tr_body(h_ref, wp_ref, bp_ref, o_ref):
  h = h_ref[...]
  h = h * lax.rsqrt(jnp.sum(h * h, axis=1, keepdims=True) + 1e-12)
  o_ref[...] = jnp.dot(h, wp_ref[...],
                       preferred_element_type=jnp.float32) + bp_ref[...]


def _norm_tr_tc(h2, Wp, bp):
  grid = (NP_ // BLK,)
  return pl.pallas_call(
      _norm_tr_body,
      grid=grid,
      in_specs=[
          pl.BlockSpec((BLK, H), lambda i: (i, 0)),
          pl.BlockSpec((H, H), lambda i: (0, 0)),
          pl.BlockSpec((1, H), lambda i: (0, 0)),
      ],
      out_specs=pl.BlockSpec((BLK, H), lambda i: (i, 0)),
      out_shape=jax.ShapeDtypeStruct((NP_, H), jnp.float32),
  )(h2, Wp.T.astype(jnp.float32), bp.reshape(1, H).astype(jnp.float32))


def _pool_body(s_ref, tr_ref, c_ref, x_ref, o_ref):
  pooled = (s_ref[...] + tr_ref[...]) / (c_ref[...] + 1.0)
  beat = x_ref[...][:, F - 1:F]
  beat16 = jnp.broadcast_to(beat, (beat.shape[0], 16))
  o_ref[...] = jnp.concatenate([pooled, beat16], axis=-1)


def _pool_tc(s_pool, tr, cpool, x_pad):
  grid = (NP_ // BLK,)
  return pl.pallas_call(
      _pool_body,
      grid=grid,
      in_specs=[
          pl.BlockSpec((BLK, H), lambda i: (i, 0)),
          pl.BlockSpec((BLK, H), lambda i: (i, 0)),
          pl.BlockSpec((BLK, 1), lambda i: (i, 0)),
          pl.BlockSpec((BLK, F), lambda i: (i, 0)),
      ],
      out_specs=pl.BlockSpec((BLK, H + 16), lambda i: (i, 0)),
      out_shape=jax.ShapeDtypeStruct((NP_, H + 16), jnp.float32),
  )(s_pool, tr, cpool, x_pad)


def _gru_cell(inp, w3, bias):
  gx = jnp.dot(inp, w3, preferred_element_type=jnp.float32)
  b = bias
  r = jax.nn.sigmoid(gx[:, 0:64] + b[:, 0:64])
  z = jax.nn.sigmoid(gx[:, 64:128] + b[:, 64:128])
  n = jnp.tanh(gx[:, 128:192] + b[:, 128:192] + r * b[:, 192:256])
  return (1.0 - z) * n


def _bn(v, g, b):
  m = jnp.mean(v, axis=0, keepdims=True)
  s = jnp.mean((v - m) * (v - m), axis=0, keepdims=True)
  return (v - m) * lax.rsqrt(s + 1e-5) * g + b


BLK2 = 2560
NB = P // BLK2


def _ep1_body(g_ref, w1_ref, b1_ref, a_ref, ps_ref):
  g = g_ref[...]
  a = jnp.dot(g, w1_ref[...], preferred_element_type=jnp.float32)
  a = jax.nn.relu(a + b1_ref[...])
  a_ref[...] = a
  ps_ref[...] = jnp.sum(a, axis=0).reshape(1, 1, a.shape[1])


def _sq_body(a_ref, ps_ref, pq_ref):
  m = jnp.sum(ps_ref[...][:, 0, :], axis=0, keepdims=True) / P
  d = a_ref[...] - m
  pq_ref[...] = jnp.sum(d * d, axis=0).reshape(1, 1, d.shape[1])


def _sq_tc(a, ps, w):
  return pl.pallas_call(
      _sq_body, grid=(NB,),
      in_specs=[pl.BlockSpec((BLK2, w), lambda i: (i, 0)),
                _full((NB, 1, w))],
      out_specs=pl.BlockSpec((1, 1, w), lambda i: (i, 0, 0)),
      out_shape=jax.ShapeDtypeStruct((NB, 1, w), jnp.float32),
  )(a, ps)


def _ep2_body(a_ref, ps_ref, pq_ref, bg_ref, bb_ref, w2_ref, b2_ref,
              b_ref, qs_ref):
  m = jnp.sum(ps_ref[...][:, 0, :], axis=0, keepdims=True) / P
  v = jnp.sum(pq_ref[...][:, 0, :], axis=0, keepdims=True) / P
  an = (a_ref[...] - m) * lax.rsqrt(v + 1e-5) * bg_ref[...] + bb_ref[...]
  b = jax.nn.relu(jnp.dot(an, w2_ref[...],
                          preferred_element_type=jnp.float32) + b2_ref[...])
  b_ref[...] = b
  qs_ref[...] = jnp.sum(b, axis=0).reshape(1, 1, b.shape[1])


def _ep3_body(b_ref, qs_ref, qq_ref, bg_ref, bb_ref,
              g0f_w_ref, g0f_b_ref, g0b_w_ref, g0b_b_ref,
              g1f_w_ref, g1f_b_ref, g1b_w_ref, g1b_b_ref,
              lng_ref, lnb_ref, o_ref):
  m = jnp.sum(qs_ref[...][:, 0, :], axis=0, keepdims=True) / P
  v = jnp.sum(qq_ref[...][:, 0, :], axis=0, keepdims=True) / P
  hb = (b_ref[...] - m) * lax.rsqrt(v + 1e-5) * bg_ref[...] + bb_ref[...]
  y = jnp.concatenate([_gru_cell(hb, g0f_w_ref[...], g0f_b_ref[...]),
                       _gru_cell(hb, g0b_w_ref[...], g0b_b_ref[...])],
                      axis=-1)
  y = jnp.concatenate([_gru_cell(y, g1f_w_ref[...], g1f_b_ref[...]),
                       _gru_cell(y, g1b_w_ref[...], g1b_b_ref[...])],
                      axis=-1)
  mu = jnp.mean(y, axis=1, keepdims=True)
  vv = jnp.mean((y - mu) * (y - mu), axis=1, keepdims=True)
  o_ref[...] = (y - mu) * lax.rsqrt(vv + 1e-5) * lng_ref[...] + lnb_ref[...]


def _full(shape):
  return pl.BlockSpec(shape, lambda i, n=len(shape): (0,) * n)


def _gru_pack(Wih, bih, bhh):
  w3 = Wih.T.astype(jnp.float32)
  Hh = 64
  bias = jnp.concatenate([
      bih[0:Hh] + bhh[0:Hh],
      bih[Hh:2 * Hh] + bhh[Hh:2 * Hh],
      bih[2 * Hh:3 * Hh],
      bhh[2 * Hh:3 * Hh],
  ]).reshape(1, 4 * Hh).astype(jnp.float32)
  return w3, bias


def _epilogue_tc(g, W1, b1, bn1_g, bn1_b, W2, b2, bn2_g, bn2_b,
                 gru0_Wih, gru0_bih, gru0_bhh,
                 gru1_Wih, gru1_bih, gru1_bhh, lng_g, lng_b):
  w1aug = jnp.zeros((H + 16, H), jnp.float32).at[:H + 1].set(
      W1.T.astype(jnp.float32))
  grid = (NB,)
  blk = lambda w: pl.BlockSpec((BLK2, w), lambda i: (i, 0))
  row = lambda w: pl.BlockSpec((1, 1, w), lambda i: (i, 0, 0))
  a, ps = pl.pallas_call(
      _ep1_body, grid=grid,
      in_specs=[blk(H + 16), _full((H + 16, H)), _full((1, H))],
      out_specs=[blk(H), row(H)],
      out_shape=[jax.ShapeDtypeStruct((P, H), jnp.float32),
                 jax.ShapeDtypeStruct((NB, 1, H), jnp.float32)],
  )(g, w1aug, b1.reshape(1, H))
  pq = _sq_tc(a, ps, H)
  Hh = H // 2
  b, qs = pl.pallas_call(
      _ep2_body, grid=grid,
      in_specs=[blk(H), _full((NB, 1, H)), _full((NB, 1, H)),
                _full((1, H)), _full((1, H)), _full((H, Hh)),
                _full((1, Hh))],
      out_specs=[blk(Hh), row(Hh)],
      out_shape=[jax.ShapeDtypeStruct((P, Hh), jnp.float32),
                 jax.ShapeDtypeStruct((NB, 1, Hh), jnp.float32)],
  )(a, ps, pq, bn1_g.reshape(1, H), bn1_b.reshape(1, H),
    W2.T.astype(jnp.float32), b2.reshape(1, Hh))
  qq = _sq_tc(b, qs, Hh)
  gargs = []
  gspecs = []
  for Wih, bih, bhh in ((gru0_Wih, gru0_bih, gru0_bhh),
                        (gru1_Wih, gru1_bih, gru1_bhh)):
    for dd in range(2):
      w3, bias = _gru_pack(Wih[dd], bih[dd], bhh[dd])
      gargs.extend([w3, bias])
      gspecs.extend([_full(w3.shape), _full(bias.shape)])
  return pl.pallas_call(
      _ep3_body, grid=grid,
      in_specs=[blk(Hh), _full((NB, 1, Hh)), _full((NB, 1, Hh)),
                _full((1, Hh)), _full((1, Hh))] + gspecs +
               [_full((1, H)), _full((1, H))],
      out_specs=blk(H),
      out_shape=jax.ShapeDtypeStruct((P, H), jnp.float32),
  )(b, qs, qq, bn2_g.reshape(1, Hh), bn2_b.reshape(1, Hh), *gargs,
    lng_g.reshape(1, H), lng_b.reshape(1, H))


def _pad_edges(ei):
  src = jnp.concatenate(
      [ei[0].astype(jnp.int32), jnp.zeros((EP - E,), jnp.int32)])
  dst = jnp.concatenate(
      [ei[1].astype(jnp.int32), jnp.full((EP - E,), N, jnp.int32)])
  return src, dst


def _chunk_table(h, n_chunks, w=16):
  h = h.at[N].set(1.0)
  return h.reshape(NP_, n_chunks, w).transpose(1, 0, 2).reshape(
      n_chunks * NP_, w)


def _uncat(s_raw, n_types, n_chunks, w=16):
  return s_raw[:n_types * n_chunks * NP_].reshape(
      n_types, n_chunks, NP_, w).transpose(
      2, 0, 1, 3).reshape(NP_, n_types * n_chunks * w)


def kernel(x, edge_onset, edge_consecutive, edge_during, edge_rest,
           edge_consecutive_rev, edge_during_rev, edge_rest_rev,
           onset_index, onset_idx, lengths, pitch_emb, spelling_emb,
           W_emb, b_emb, enc0_Wr, enc0_Wn, enc0_b, enc1_Wr, enc1_Wn,
           enc1_b, Wp, bp, W1, b1, bn1_g, bn1_b, W2, b2, bn2_g, bn2_b,
           gru0_Wih, gru0_Whh, gru0_bih, gru0_bhh, gru1_Wih, gru1_Whh,
           gru1_bih, gru1_bhh, lng_g, lng_b):
  edges = [edge_onset, edge_consecutive, edge_during, edge_rest,
           edge_consecutive_rev, edge_during_rev, edge_rest_rev]
  srcs, dsts = zip(*[_pad_edges(e) for e in edges])
  psrc, pdst = _pad_edges(onset_index)
  src7 = jnp.concatenate(srcs)
  dst7 = jnp.concatenate(dsts)
  dst8 = jnp.concatenate([dst7, pdst])

  x_pad = jnp.zeros((NP_, F), jnp.float32).at[:N].set(
      x.astype(jnp.float32))

  emb_tab = jnp.concatenate([pitch_emb.astype(jnp.float32),
                             spelling_emb.astype(jnp.float32)])
  eidx = jnp.concatenate([x_pad[:, 0].astype(jnp.int32),
                          128 + x_pad[:, 1].astype(jnp.int32)])
  hp_hs = _gather_sc(emb_tab, eidx, 2 * NP_, 112)
  h0 = _embed_tc(x_pad, hp_hs[:NP_], hp_hs[NP_:], W_emb, b_emb)

  src_cnt = jnp.full((EP,), N, jnp.int32)
  src15 = jnp.concatenate(list(srcs) + [src_cnt] * 8)
  dst15 = jnp.concatenate(list(dsts) + list(dsts) + [pdst])
  s0 = _seg_sum_sc(_chunk_table(h0, 2, 32), src15, dst15, 7, 2, n_cnt=8,
                   w=32)
  cnt = s0[7 * 2 * NP_:].reshape(8, 2, NP_, 32)[:, 0, :, 0]
  cnt7_t = cnt[:7].T
  cpool = cnt[7].reshape(NP_, 1)
  h1 = _layer_tc(_uncat(s0, 7, 2, 32), cnt7_t, h0, enc0_Wn, enc0_Wr,
                 enc0_b)

  s1 = _seg_sum_sc(_chunk_table(h1, 4, 32), src7, dst7, 7, 4, w=32)
  h2 = _layer_tc(_uncat(s1, 7, 4, 32), cnt7_t, h1, enc1_Wn, enc1_Wr,
                 enc1_b)

  tr = _norm_tr_tc(h2, Wp, bp)

  sp = _seg_sum_sc(_chunk_table(tr, 8), psrc, pdst, 1, 8)
  s_pool = _uncat(sp, 1, 8)
  tbl = _pool_tc(s_pool, tr, cpool, x_pad)

  g = _gather_sc(tbl, onset_idx.astype(jnp.int32), P, 80)

  return _epilogue_tc(g, W1, b1, bn1_g, bn1_b, W2, b2, bn2_g, bn2_b,
                      gru0_Wih, gru0_bih, gru0_bhh,
                      gru1_Wih, gru1_bih, gru1_bhh, lng_g, lng_b)


# speedup vs baseline: 1.2939x; 1.2939x over previous
import functools

import jax
import jax.numpy as jnp
from jax import lax
from jax.experimental import pallas as pl
from jax.experimental.pallas import tpu as pltpu
from jax.experimental.pallas import tpu_sc as plsc

N = 50000
E = 80000
P = 25600
F = 20
H = 128

NP_ = 50176
EP = 81920
SUB = 128
TILE_E = EP // 16
NSUB = TILE_E // SUB
HALF = NP_ // 2
ACC_R = HALF + 64
TILE_R = HALF // 16
ZR = 784

_mesh = functools.partial(
    plsc.VectorSubcoreMesh, core_axis_name="c", subcore_axis_name="s")


def _seg_sum_sc(table, src_flat, dst_flat, n_real, n_chunks, n_cnt=0,
                w=16):
  @functools.partial(
      pl.kernel, mesh=_mesh(),
      compiler_params=pltpu.CompilerParams(use_tc_tiling_on_sc=False),
      out_type=jax.ShapeDtypeStruct(((n_real + n_cnt) * n_chunks * NP_, w),
                                    jnp.float32),
      scratch_types=[
          pltpu.VMEM((SUB,), jnp.int32),
          pltpu.VMEM((SUB,), jnp.int32),
          pltpu.VMEM((SUB,), jnp.int32),
          pltpu.VMEM((SUB,), jnp.int32),
          pltpu.VMEM((SUB, w), jnp.float32),
          pltpu.VMEM((SUB, w), jnp.float32),
          pltpu.VMEM((ZR, w), jnp.float32),
          pltpu.VMEM((ZR, w), jnp.float32),
          pltpu.SemaphoreType.DMA,
          pltpu.SemaphoreType.DMA,
          pltpu.VMEM_SHARED((ACC_R, w), jnp.float32),
      ])
  def seg(table_hbm, src_hbm, dst_hbm, zero_hbm, out_hbm,
          src_v0, src_v1, dst_v0, dst_v1, rows_v0, rows_v1,
          zero_v, drain_v, sem0, sem1, acc):
    c = lax.axis_index("c")
    s = lax.axis_index("s")
    pltpu.sync_copy(zero_hbm, zero_v)
    row0 = s * TILE_R
    half0 = c * HALF
    srcs = (src_v0, src_v1)
    dsts = (dst_v0, dst_v1)
    rows = (rows_v0, rows_v1)
    sems = (sem0, sem1)
    for chunk in range(n_chunks):
      tab_base = chunk * NP_
      n_t = n_real + n_cnt if chunk == 0 else n_real

      def tbody(t, _):
        for r in range(TILE_R // ZR):
          pltpu.sync_copy(zero_v, acc.at[pl.ds(row0 + r * ZR, ZR)])
        plsc.subcore_barrier()
        ebase0 = t * EP + s * TILE_E

        def ebody(jj, _):
          cps = []
          for b in range(2):
            eb = ebase0 + (jj * 2 + b) * SUB
            pltpu.sync_copy(src_hbm.at[pl.ds(eb, SUB)], srcs[b])
            pltpu.sync_copy(dst_hbm.at[pl.ds(eb, SUB)], dsts[b])
            for q in range(SUB // 16):
              sl = pl.ds(q * 16, 16)
              if chunk:
                srcs[b][sl] = srcs[b][sl] + tab_base
              dl = dsts[b][sl] - half0
              ok = (dl >= 0) & (dl < HALF)
              dsts[b][sl] = jnp.where(ok, dl, HALF)
            cps.append(pltpu.async_copy(table_hbm.at[srcs[b]], rows[b],
                                        sems[b]))
          for b in range(2):
            cps[b].wait()
            pltpu.sync_copy(rows[b], acc.at[dsts[b]], add=True)
          return 0

        lax.fori_loop(0, NSUB // 2, ebody, 0)
        plsc.subcore_barrier()
        obase = (t * n_chunks + chunk) * NP_ + half0 + row0
        for r in range(TILE_R // ZR):
          pltpu.sync_copy(acc.at[pl.ds(row0 + r * ZR, ZR)], drain_v)
          pltpu.sync_copy(drain_v, out_hbm.at[pl.ds(obase + r * ZR, ZR)])
        plsc.subcore_barrier()
        return 0

      lax.fori_loop(0, n_t, tbody, 0)

  zero_hbm = jnp.zeros((ZR, w), jnp.float32)
  return seg(table, src_flat, dst_flat, zero_hbm)


def _gather_sc(table, idx, n_rows, gsub):
  bpw = n_rows // 32
  D = table.shape[1]

  @functools.partial(
      pl.kernel, mesh=_mesh(),
      compiler_params=pltpu.CompilerParams(use_tc_tiling_on_sc=False),
      out_type=jax.ShapeDtypeStruct((n_rows, D), jnp.float32),
      scratch_types=[
          pltpu.VMEM((gsub,), jnp.int32),
          pltpu.VMEM((gsub, D), jnp.float32),
          pltpu.SemaphoreType.DMA,
      ])
  def gat(table_hbm, idx_hbm, out_hbm, idx_v, rows_v, sem):
    c = lax.axis_index("c")
    s = lax.axis_index("s")
    wid = s * 2 + c
    base = wid * bpw

    def body(j, _):
      b = base + j * gsub
      pltpu.sync_copy(idx_hbm.at[pl.ds(b, gsub)], idx_v)
      pltpu.async_copy(table_hbm.at[idx_v], rows_v, sem).wait()
      pltpu.sync_copy(rows_v, out_hbm.at[pl.ds(b, gsub)])
      return 0

    lax.fori_loop(0, bpw // gsub, body, 0)

  return gat(table, idx)


BLK = 256


def _embed_body(x_ref, hp_ref, hs_ref, we_ref, be_ref, o_ref):
  xb = x_ref[...]
  hc = jnp.dot(xb[:, 2:F - 1], we_ref[...],
               preferred_element_type=jnp.float32) + be_ref[...]
  o_ref[...] = jnp.concatenate([hc, hp_ref[...], hs_ref[...]], axis=-1)


def _embed_tc(x_pad, hp, hs, W_emb, b_emb):
  wet = W_emb.T.astype(jnp.float32)
  be = b_emb.reshape(1, 32).astype(jnp.float32)
  grid = (NP_ // BLK,)
  return pl.pallas_call(
      _embed_body,
      grid=grid,
      in_specs=[
          pl.BlockSpec((BLK, F), lambda i: (i, 0)),
          pl.BlockSpec((BLK, 16), lambda i: (i, 0)),
          pl.BlockSpec((BLK, 16), lambda i: (i, 0)),
          pl.BlockSpec((F - 3, 32), lambda i: (0, 0)),
          pl.BlockSpec((1, 32), lambda i: (0, 0)),
      ],
      out_specs=pl.BlockSpec((BLK, 64), lambda i: (i, 0)),
      out_shape=jax.ShapeDtypeStruct((NP_, 64), jnp.float32),
  )(x_pad, hp, hs, wet, be)


def _layer_body(s_ref, cnt_ref, h_ref, w_ref, b_ref, o_ref, *,
                n_types, d):
  sblk = s_ref[...]
  blk = sblk.shape[0]
  inv = 1.0 / jnp.maximum(cnt_ref[...], 1.0)
  m = sblk.reshape(blk, n_types, d) * inv[:, :, None]
  h = h_ref[...]
  hrep = jnp.broadcast_to(h[:, None, :], (blk, n_types, d))
  maug = jnp.concatenate([m, hrep], axis=1).reshape(blk, 2 * n_types * d)
  acc = jnp.dot(maug, w_ref[...], preferred_element_type=jnp.float32)
  o_ref[...] = jax.nn.relu((acc + b_ref[...]) / float(n_types))


def _layer_tc(s_cat, cnt_t, h_in, Wn, Wr, b):
  n_types = Wn.shape[0]
  d = h_in.shape[1]
  w_aug = jnp.concatenate([
      Wn.transpose(0, 2, 1).reshape(n_types * d, H),
      Wr.transpose(0, 2, 1).reshape(n_types * d, H)], axis=0)
  b_eff = b.sum(0).reshape(1, H).astype(jnp.float32)
  grid = (NP_ // BLK,)
  return pl.pallas_call(
      functools.partial(_layer_body, n_types=n_types, d=d),
      grid=grid,
      in_specs=[
          pl.BlockSpec((BLK, n_types * d), lambda i: (i, 0)),
          pl.BlockSpec((BLK, n_types), lambda i: (i, 0)),
          pl.BlockSpec((BLK, d), lambda i: (i, 0)),
          pl.BlockSpec((2 * n_types * d, H), lambda i: (0, 0)),
          pl.BlockSpec((1, H), lambda i: (0, 0)),
      ],
      out_specs=pl.BlockSpec((BLK, H), lambda i: (i, 0)),
      out_shape=jax.ShapeDtypeStruct((NP_, H), jnp.float32),
  )(s_cat, cnt_t, h_in, w_aug.astype(jnp.float32), b_eff)


def _norm_tr_body(h_ref, wp_ref, bp_ref, o_ref):
  h = h_ref[...]
  h = h * lax.rsqrt(jnp.sum(h * h, axis=1, keepdims=True) + 1e-12)
  o_ref[...] = jnp.dot(h, wp_ref[...],
                       preferred_element_type=jnp.float32) + bp_ref[...]


def _norm_tr_tc(h2, Wp, bp):
  grid = (NP_ // BLK,)
  return pl.pallas_call(
      _norm_tr_body,
      grid=grid,
      in_specs=[
          pl.BlockSpec((BLK, H), lambda i: (i, 0)),
          pl.BlockSpec((H, H), lambda i: (0, 0)),
          pl.BlockSpec((1, H), lambda i: (0, 0)),
      ],
      out_specs=pl.BlockSpec((BLK, H), lambda i: (i, 0)),
      out_shape=jax.ShapeDtypeStruct((NP_, H), jnp.float32),
  )(h2, Wp.T.astype(jnp.float32), bp.reshape(1, H).astype(jnp.float32))


def _pool_body(s_ref, tr_ref, c_ref, x_ref, o_ref):
  pooled = (s_ref[...] + tr_ref[...]) / (c_ref[...] + 1.0)
  beat = x_ref[...][:, F - 1:F]
  beat16 = jnp.broadcast_to(beat, (beat.shape[0], 16))
  o_ref[...] = jnp.concatenate([pooled, beat16], axis=-1)


def _pool_tc(s_pool, tr, cpool, x_pad):
  grid = (NP_ // BLK,)
  return pl.pallas_call(
      _pool_body,
      grid=grid,
      in_specs=[
          pl.BlockSpec((BLK, H), lambda i: (i, 0)),
          pl.BlockSpec((BLK, H), lambda i: (i, 0)),
          pl.BlockSpec((BLK, 1), lambda i: (i, 0)),
          pl.BlockSpec((BLK, F), lambda i: (i, 0)),
      ],
      out_specs=pl.BlockSpec((BLK, H + 16), lambda i: (i, 0)),
      out_shape=jax.ShapeDtypeStruct((NP_, H + 16), jnp.float32),
  )(s_pool, tr, cpool, x_pad)


def _gru_cell(inp, w3, bias):
  gx = jnp.dot(inp, w3, preferred_element_type=jnp.float32)
  b = bias
  r = jax.nn.sigmoid(gx[:, 0:64] + b[:, 0:64])
  z = jax.nn.sigmoid(gx[:, 64:128] + b[:, 64:128])
  n = jnp.tanh(gx[:, 128:192] + b[:, 128:192] + r * b[:, 192:256])
  return (1.0 - z) * n


def _bn(v, g, b):
  m = jnp.mean(v, axis=0, keepdims=True)
  s = jnp.mean((v - m) * (v - m), axis=0, keepdims=True)
  return (v - m) * lax.rsqrt(s + 1e-5) * g + b


BLK2 = 2560
NB = P // BLK2


def _ep1_body(g_ref, w1_ref, b1_ref, a_ref, ps_ref):
  g = g_ref[...]
  a = jnp.dot(g, w1_ref[...], preferred_element_type=jnp.float32)
  a = jax.nn.relu(a + b1_ref[...])
  a_ref[...] = a
  ps_ref[...] = jnp.sum(a, axis=0).reshape(1, 1, a.shape[1])


def _sq_body(a_ref, ps_ref, pq_ref):
  m = jnp.sum(ps_ref[...][:, 0, :], axis=0, keepdims=True) / P
  d = a_ref[...] - m
  pq_ref[...] = jnp.sum(d * d, axis=0).reshape(1, 1, d.shape[1])


def _sq_tc(a, ps, w):
  return pl.pallas_call(
      _sq_body, grid=(NB,),
      in_specs=[pl.BlockSpec((BLK2, w), lambda i: (i, 0)),
                _full((NB, 1, w))],
      out_specs=pl.BlockSpec((1, 1, w), lambda i: (i, 0, 0)),
      out_shape=jax.ShapeDtypeStruct((NB, 1, w), jnp.float32),
  )(a, ps)


def _ep2_body(a_ref, ps_ref, pq_ref, bg_ref, bb_ref, w2_ref, b2_ref,
              b_ref, qs_ref):
  m = jnp.sum(ps_ref[...][:, 0, :], axis=0, keepdims=True) / P
  v = jnp.sum(pq_ref[...][:, 0, :], axis=0, keepdims=True) / P
  an = (a_ref[...] - m) * lax.rsqrt(v + 1e-5) * bg_ref[...] + bb_ref[...]
  b = jax.nn.relu(jnp.dot(an, w2_ref[...],
                          preferred_element_type=jnp.float32) + b2_ref[...])
  b_ref[...] = b
  qs_ref[...] = jnp.sum(b, axis=0).reshape(1, 1, b.shape[1])


def _ep3_body(b_ref, qs_ref, qq_ref, bg_ref, bb_ref,
              g0f_w_ref, g0f_b_ref, g0b_w_ref, g0b_b_ref,
              g1f_w_ref, g1f_b_ref, g1b_w_ref, g1b_b_ref,
              lng_ref, lnb_ref, o_ref):
  m = jnp.sum(qs_ref[...][:, 0, :], axis=0, keepdims=True) / P
  v = jnp.sum(qq_ref[...][:, 0, :], axis=0, keepdims=True) / P
  hb = (b_ref[...] - m) * lax.rsqrt(v + 1e-5) * bg_ref[...] + bb_ref[...]
  y = jnp.concatenate([_gru_cell(hb, g0f_w_ref[...], g0f_b_ref[...]),
                       _gru_cell(hb, g0b_w_ref[...], g0b_b_ref[...])],
                      axis=-1)
  y = jnp.concatenate([_gru_cell(y, g1f_w_ref[...], g1f_b_ref[...]),
                       _gru_cell(y, g1b_w_ref[...], g1b_b_ref[...])],
                      axis=-1)
  mu = jnp.mean(y, axis=1, keepdims=True)
  vv = jnp.mean((y - mu) * (y - mu), axis=1, keepdims=True)
  o_ref[...] = (y - mu) * lax.rsqrt(vv + 1e-5) * lng_ref[...] + lnb_ref[...]


def _full(shape):
  return pl.BlockSpec(shape, lambda i, n=len(shape): (0,) * n)


def _gru_pack(Wih, bih, bhh):
  w3 = Wih.T.astype(jnp.float32)
  Hh = 64
  bias = jnp.concatenate([
      bih[0:Hh] + bhh[0:Hh],
      bih[Hh:2 * Hh] + bhh[Hh:2 * Hh],
      bih[2 * Hh:3 * Hh],
      bhh[2 * Hh:3 * Hh],
  ]).reshape(1, 4 * Hh).astype(jnp.float32)
  return w3, bias


def _epilogue_tc(g, W1, b1, bn1_g, bn1_b, W2, b2, bn2_g, bn2_b,
                 gru0_Wih, gru0_bih, gru0_bhh,
                 gru1_Wih, gru1_bih, gru1_bhh, lng_g, lng_b):
  w1aug = jnp.zeros((H + 16, H), jnp.float32).at[:H + 1].set(
      W1.T.astype(jnp.float32))
  grid = (NB,)
  blk = lambda w: pl.BlockSpec((BLK2, w), lambda i: (i, 0))
  row = lambda w: pl.BlockSpec((1, 1, w), lambda i: (i, 0, 0))
  a, ps = pl.pallas_call(
      _ep1_body, grid=grid,
      in_specs=[blk(H + 16), _full((H + 16, H)), _full((1, H))],
      out_specs=[blk(H), row(H)],
      out_shape=[jax.ShapeDtypeStruct((P, H), jnp.float32),
                 jax.ShapeDtypeStruct((NB, 1, H), jnp.float32)],
  )(g, w1aug, b1.reshape(1, H))
  pq = _sq_tc(a, ps, H)
  Hh = H // 2
  b, qs = pl.pallas_call(
      _ep2_body, grid=grid,
      in_specs=[blk(H), _full((NB, 1, H)), _full((NB, 1, H)),
                _full((1, H)), _full((1, H)), _full((H, Hh)),
                _full((1, Hh))],
      out_specs=[blk(Hh), row(Hh)],
      out_shape=[jax.ShapeDtypeStruct((P, Hh), jnp.float32),
                 jax.ShapeDtypeStruct((NB, 1, Hh), jnp.float32)],
  )(a, ps, pq, bn1_g.reshape(1, H), bn1_b.reshape(1, H),
    W2.T.astype(jnp.float32), b2.reshape(1, Hh))
  qq = _sq_tc(b, qs, Hh)
  gargs = []
  gspecs = []
  for Wih, bih, bhh in ((gru0_Wih, gru0_bih, gru0_bhh),
                        (gru1_Wih, gru1_bih, gru1_bhh)):
    for dd in range(2):
      w3, bias = _gru_pack(Wih[dd], bih[dd], bhh[dd])
      gargs.extend([w3, bias])
      gspecs.extend([_full(w3.shape), _full(bias.shape)])
  return pl.pallas_call(
      _ep3_body, grid=grid,
      in_specs=[blk(Hh), _full((NB, 1, Hh)), _full((NB, 1, Hh)),
                _full((1, Hh)), _full((1, Hh))] + gspecs +
               [_full((1, H)), _full((1, H))],
      out_specs=blk(H),
      out_shape=jax.ShapeDtypeStruct((P, H), jnp.float32),
  )(b, qs, qq, bn2_g.reshape(1, Hh), bn2_b.reshape(1, Hh), *gargs,
    lng_g.reshape(1, H), lng_b.reshape(1, H))


def _pad_edges(ei):
  src = jnp.concatenate(
      [ei[0].astype(jnp.int32), jnp.zeros((EP - E,), jnp.int32)])
  dst = jnp.concatenate(
      [ei[1].astype(jnp.int32), jnp.full((EP - E,), N, jnp.int32)])
  return src, dst


def _chunk_table(h, n_chunks, w=16):
  h = h.at[N].set(1.0)
  return h.reshape(NP_, n_chunks, w).transpose(1, 0, 2).reshape(
      n_chunks * NP_, w)


def _uncat(s_raw, n_types, n_chunks, w=16):
  return s_raw[:n_types * n_chunks * NP_].reshape(
      n_types, n_chunks, NP_, w).transpose(
      2, 0, 1, 3).reshape(NP_, n_types * n_chunks * w)


def kernel(x, edge_onset, edge_consecutive, edge_during, edge_rest,
           edge_consecutive_rev, edge_during_rev, edge_rest_rev,
           onset_index, onset_idx, lengths, pitch_emb, spelling_emb,
           W_emb, b_emb, enc0_Wr, enc0_Wn, enc0_b, enc1_Wr, enc1_Wn,
           enc1_b, Wp, bp, W1, b1, bn1_g, bn1_b, W2, b2, bn2_g, bn2_b,
           gru0_Wih, gru0_Whh, gru0_bih, gru0_bhh, gru1_Wih, gru1_Whh,
           gru1_bih, gru1_bhh, lng_g, lng_b):
  edges = [edge_onset, edge_consecutive, edge_during, edge_rest,
           edge_consecutive_rev, edge_during_rev, edge_rest_rev]
  srcs, dsts = zip(*[_pad_edges(e) for e in edges])
  psrc, pdst = _pad_edges(onset_index)
  src7 = jnp.concatenate(srcs)
  dst7 = jnp.concatenate(dsts)
  dst8 = jnp.concatenate([dst7, pdst])

  x_pad = jnp.zeros((NP_, F), jnp.float32).at[:N].set(
      x.astype(jnp.float32))

  emb_tab = jnp.concatenate([pitch_emb.astype(jnp.float32),
                             spelling_emb.astype(jnp.float32)])
  eidx = jnp.concatenate([x_pad[:, 0].astype(jnp.int32),
                          128 + x_pad[:, 1].astype(jnp.int32)])
  hp_hs = _gather_sc(emb_tab, eidx, 2 * NP_, 112)
  h0 = _embed_tc(x_pad, hp_hs[:NP_], hp_hs[NP_:], W_emb, b_emb)

  src_cnt = jnp.full((EP,), N, jnp.int32)
  src15 = jnp.concatenate(list(srcs) + [src_cnt] * 8)
  dst15 = jnp.concatenate(list(dsts) + list(dsts) + [pdst])
  s0 = _seg_sum_sc(_chunk_table(h0, 4), src15, dst15, 7, 4, n_cnt=8)
  cnt = s0[7 * 4 * NP_:].reshape(8, 4, NP_, 16)[:, 0, :, 0]
  cnt7_t = cnt[:7].T
  cpool = cnt[7].reshape(NP_, 1)
  h1 = _layer_tc(_uncat(s0, 7, 4), cnt7_t, h0, enc0_Wn, enc0_Wr, enc0_b)

  s1 = _seg_sum_sc(_chunk_table(h1, 4, 32), src7, dst7, 7, 4, w=32)
  h2 = _layer_tc(_uncat(s1, 7, 4, 32), cnt7_t, h1, enc1_Wn, enc1_Wr,
                 enc1_b)

  tr = _norm_tr_tc(h2, Wp, bp)

  sp = _seg_sum_sc(_chunk_table(tr, 8), psrc, pdst, 1, 8)
  s_pool = _uncat(sp, 1, 8)
  tbl = _pool_tc(s_pool, tr, cpool, x_pad)

  g = _gather_sc(tbl, onset_idx.astype(jnp.int32), P, 80)

  return _epilogue_tc(g, W1, b1, bn1_g, bn1_b, W2, b2, bn2_g, bn2_b,
                      gru0_Wih, gru0_bih, gru0_bhh,
                      gru1_Wih, gru1_bih, gru1_bhh, lng_g, lng_b)
